# baseline (speedup 1.0000x reference)
import functools

import jax
import jax.numpy as jnp
from jax import lax
from jax.experimental import pallas as pl
from jax.experimental.pallas import tpu as pltpu

C = 128


def _body(map_ref, act_ref, x_ref, w1_ref, b1_ref, w2_ref, b2_ref,
          tok_out_ref, act_out_ref, D, A, *, tn, nt, h, ns):
    t = pl.program_id(1)

    @pl.when(t == 0)
    def _init():
        D[...] = jnp.zeros((C, 1), jnp.float32)
        A[...] = jnp.zeros((C, h), jnp.float32)

    sn = tn // ns
    ci = lax.broadcasted_iota(jnp.int16, (C, sn), 0)
    d_parts = []
    a_parts = []
    for s in range(ns):
        x = x_ref[0, pl.ds(s * sn, sn), :]
        chan_row = map_ref[0, :, pl.ds(s * sn, sn)]
        act_row = act_ref[0, :, pl.ds(s * sn, sn)]

        xb = x.astype(jnp.bfloat16)
        h_t = jnp.tanh(
            lax.dot_general(w1_ref[...], xb, (((1,), (1,)), ((), ())),
                            preferred_element_type=jnp.float32)
            + b1_ref[...])
        g_row = (jnp.dot(w2_ref[...], h_t.astype(jnp.bfloat16),
                         preferred_element_type=jnp.float32)
                 + b2_ref[...])
        p_row = jnp.exp(jnp.clip(g_row, -40.0, 40.0)) * act_row
        p_row_bf = p_row.astype(jnp.bfloat16)

        p = jnp.where(chan_row == ci, p_row_bf,
                      jnp.bfloat16(0.0))

        d_parts.append(jnp.sum(p, axis=1, keepdims=True,
                               dtype=jnp.float32))
        a_parts.append(jnp.dot(p, xb, preferred_element_type=jnp.float32))

    D[...] += sum(d_parts)
    A[...] += sum(a_parts)

    @pl.when(t == nt - 1)
    def _finish():
        d = D[...]
        tok_out_ref[0] = A[...] / jnp.maximum(d, 1e-30)
        act_out_ref[0] = (d > 0.0).astype(jnp.float32)


def kernel(block_tokens, block_active, block_to_channel_map, W1, b1, W2, b2):
    B, NB, H = block_tokens.shape
    K = W1.shape[0]
    TN = 2048
    NS = 4
    NT = NB // TN

    map3 = block_to_channel_map.astype(jnp.int16).reshape(1, 1, NB)
    act3 = block_active.astype(jnp.float32).reshape(B, 1, NB)
    b1c = b1.reshape(K, 1)
    b2c = jnp.asarray(b2).reshape(1, 1)

    grid = (B, NT)
    out_tok, out_act = pl.pallas_call(
        functools.partial(_body, tn=TN, nt=NT, h=H, ns=NS),
        grid=grid,
        in_specs=[
            pl.BlockSpec((1, 1, TN), lambda b, t: (0, 0, t)),
            pl.BlockSpec((1, 1, TN), lambda b, t: (b, 0, t)),
            pl.BlockSpec((1, TN, H), lambda b, t: (b, t, 0)),
            pl.BlockSpec((K, H), lambda b, t: (0, 0)),
            pl.BlockSpec((K, 1), lambda b, t: (0, 0)),
            pl.BlockSpec((1, K), lambda b, t: (0, 0)),
            pl.BlockSpec((1, 1), lambda b, t: (0, 0)),
        ],
        out_specs=[
            pl.BlockSpec((1, C, H), lambda b, t: (b, 0, 0)),
            pl.BlockSpec((1, C, 1), lambda b, t: (b, 0, 0)),
        ],
        out_shape=[
            jax.ShapeDtypeStruct((B, C, H), jnp.float32),
            jax.ShapeDtypeStruct((B, C, 1), jnp.float32),
        ],
        scratch_shapes=[
            pltpu.VMEM((C, 1), jnp.float32),
            pltpu.VMEM((C, H), jnp.float32),
        ],
        compiler_params=pltpu.CompilerParams(
            dimension_semantics=("parallel", "arbitrary")),
    )(map3, act3, block_tokens, W1.astype(jnp.bfloat16), b1c,
      W2.astype(jnp.bfloat16), b2c)

    return out_tok, out_act.reshape(B, C) > 0.0


# speedup vs baseline: 1.3756x; 1.3756x over previous
import functools

import jax
import jax.numpy as jnp
from jax import lax
from jax.experimental import pallas as pl
from jax.experimental.pallas import tpu as pltpu

C = 128


def _body(map_ref, act_ref, x_ref, w1_ref, b1_ref, w2_ref, b2_ref,
          tok_out_ref, act_out_ref, D, A, *, tn, nt, h, ns):
    t = pl.program_id(1)

    @pl.when(t == 0)
    def _init():
        D[...] = jnp.zeros((C, 1), jnp.float32)
        A[...] = jnp.zeros((C, h), jnp.float32)

    ci = lax.broadcasted_iota(jnp.int16, (C, tn), 0)
    x = x_ref[0]
    chan_row = map_ref[0]
    act_row = act_ref[0]

    xb = x.astype(jnp.bfloat16)
    h_t = jnp.tanh(
        lax.dot_general(w1_ref[...], xb, (((1,), (1,)), ((), ())),
                        preferred_element_type=jnp.float32)
        + b1_ref[...])
    g_row = (jnp.dot(w2_ref[...], h_t.astype(jnp.bfloat16),
                     preferred_element_type=jnp.float32)
             + b2_ref[...])
    p_row = jnp.exp(jnp.clip(g_row, -40.0, 40.0)) * act_row
    p_row_bf = p_row.astype(jnp.bfloat16)

    p = jnp.where(chan_row == ci, p_row_bf,
                  jnp.bfloat16(0.0))

    D[...] += jnp.sum(p, axis=1, keepdims=True, dtype=jnp.float32)
    A[...] += jnp.dot(p, xb, preferred_element_type=jnp.float32)

    @pl.when(t == nt - 1)
    def _finish():
        d = D[...]
        tok_out_ref[0] = A[...] / jnp.maximum(d, 1e-30)
        act_out_ref[0] = (d > 0.0).astype(jnp.float32)


def kernel(block_tokens, block_active, block_to_channel_map, W1, b1, W2, b2):
    B, NB, H = block_tokens.shape
    K = W1.shape[0]
    TN = 2048
    NS = 4
    NT = NB // TN

    map3 = block_to_channel_map.astype(jnp.int16).reshape(1, 1, NB)
    act3 = block_active.astype(jnp.float32).reshape(B, 1, NB)
    b1c = b1.reshape(K, 1)
    b2c = jnp.asarray(b2).reshape(1, 1)

    grid = (B, NT)
    out_tok, out_act = pl.pallas_call(
        functools.partial(_body, tn=TN, nt=NT, h=H, ns=NS),
        grid=grid,
        in_specs=[
            pl.BlockSpec((1, 1, TN), lambda b, t: (0, 0, t)),
            pl.BlockSpec((1, 1, TN), lambda b, t: (b, 0, t)),
            pl.BlockSpec((1, TN, H), lambda b, t: (b, t, 0)),
            pl.BlockSpec((K, H), lambda b, t: (0, 0)),
            pl.BlockSpec((K, 1), lambda b, t: (0, 0)),
            pl.BlockSpec((1, K), lambda b, t: (0, 0)),
            pl.BlockSpec((1, 1), lambda b, t: (0, 0)),
        ],
        out_specs=[
            pl.BlockSpec((1, C, H), lambda b, t: (b, 0, 0)),
            pl.BlockSpec((1, C, 1), lambda b, t: (b, 0, 0)),
        ],
        out_shape=[
            jax.ShapeDtypeStruct((B, C, H), jnp.float32),
            jax.ShapeDtypeStruct((B, C, 1), jnp.float32),
        ],
        scratch_shapes=[
            pltpu.VMEM((C, 1), jnp.float32),
            pltpu.VMEM((C, H), jnp.float32),
        ],
        compiler_params=pltpu.CompilerParams(
            dimension_semantics=("parallel", "arbitrary")),
    )(map3, act3, block_tokens, W1.astype(jnp.bfloat16), b1c,
      W2.astype(jnp.bfloat16), b2c)

    return out_tok, out_act.reshape(B, C) > 0.0


# speedup vs baseline: 2.2391x; 1.6277x over previous
import functools

import jax
import jax.numpy as jnp
from jax import lax
from jax.experimental import pallas as pl
from jax.experimental.pallas import tpu as pltpu

C = 128


def _body(map_ref, act_ref, x_ref, w1_ref, b1_ref, w2_ref, b2_ref,
          tok_out_ref, act_out_ref, D, A, *, tn, nt, h, ns):
    t = pl.program_id(1)

    @pl.when(t == 0)
    def _init():
        D[...] = jnp.zeros((C, 1), jnp.float32)
        A[...] = jnp.zeros((C, h), jnp.float32)

    sn = tn // ns
    ci = lax.broadcasted_iota(jnp.int16, (C, sn), 0)
    rng = range(ns)
    xb = [x_ref[0, pl.ds(s * sn, sn), :].astype(jnp.bfloat16) for s in rng]
    pre = [lax.dot_general(w1_ref[...], xb[s], (((1,), (1,)), ((), ())),
                           preferred_element_type=jnp.float32) for s in rng]
    h_t = [jnp.tanh(pre[s] + b1_ref[...]).astype(jnp.bfloat16) for s in rng]
    g = [jnp.dot(w2_ref[...], h_t[s], preferred_element_type=jnp.float32)
         + b2_ref[...] for s in rng]
    p_row = [(jnp.exp(jnp.clip(g[s], -40.0, 40.0))
              * act_ref[0, :, pl.ds(s * sn, sn)]).astype(jnp.bfloat16)
             for s in rng]
    p = [jnp.where(map_ref[0, :, pl.ds(s * sn, sn)] == ci, p_row[s],
                   jnp.bfloat16(0.0)) for s in rng]
    d = [jnp.sum(p[s], axis=1, keepdims=True, dtype=jnp.float32)
         for s in rng]
    a = [jnp.dot(p[s], xb[s], preferred_element_type=jnp.float32)
         for s in rng]

    D[...] += sum(d)
    A[...] += sum(a)

    @pl.when(t == nt - 1)
    def _finish():
        dd = D[...]
        tok_out_ref[0] = A[...] / jnp.maximum(dd, 1e-30)
        act_out_ref[0] = (dd > 0.0).astype(jnp.float32)


def kernel(block_tokens, block_active, block_to_channel_map, W1, b1, W2, b2):
    B, NB, H = block_tokens.shape
    K = W1.shape[0]
    TN = 8192
    NS = 1
    NT = NB // TN

    map3 = block_to_channel_map.astype(jnp.int16).reshape(1, 1, NB)
    act3 = block_active.astype(jnp.float32).reshape(B, 1, NB)
    b1c = b1.reshape(K, 1)
    b2c = jnp.asarray(b2).reshape(1, 1)

    grid = (B, NT)
    out_tok, out_act = pl.pallas_call(
        functools.partial(_body, tn=TN, nt=NT, h=H, ns=NS),
        grid=grid,
        in_specs=[
            pl.BlockSpec((1, 1, TN), lambda b, t: (0, 0, t)),
            pl.BlockSpec((1, 1, TN), lambda b, t: (b, 0, t)),
            pl.BlockSpec((1, TN, H), lambda b, t: (b, t, 0)),
            pl.BlockSpec((K, H), lambda b, t: (0, 0)),
            pl.BlockSpec((K, 1), lambda b, t: (0, 0)),
            pl.BlockSpec((1, K), lambda b, t: (0, 0)),
            pl.BlockSpec((1, 1), lambda b, t: (0, 0)),
        ],
        out_specs=[
            pl.BlockSpec((1, C, H), lambda b, t: (b, 0, 0)),
            pl.BlockSpec((1, C, 1), lambda b, t: (b, 0, 0)),
        ],
        out_shape=[
            jax.ShapeDtypeStruct((B, C, H), jnp.float32),
            jax.ShapeDtypeStruct((B, C, 1), jnp.float32),
        ],
        scratch_shapes=[
            pltpu.VMEM((C, 1), jnp.float32),
            pltpu.VMEM((C, H), jnp.float32),
        ],
        compiler_params=pltpu.CompilerParams(
            dimension_semantics=("parallel", "arbitrary")),
    )(map3, act3, block_tokens, W1.astype(jnp.bfloat16), b1c,
      W2.astype(jnp.bfloat16), b2c)

    return out_tok, out_act.reshape(B, C) > 0.0
